# Initial kernel scaffold; baseline (speedup 1.0000x reference)
import jax
import jax.numpy as jnp
from jax.experimental import pallas as pl


def kernel(input, hx, weight_ih, weight_hh, bias_ih, bias_hh, memory):
    raise NotImplementedError("write your pallas kernel here")


# speedup vs baseline: 12.6886x; 12.6886x over previous
import functools

import jax
import jax.numpy as jnp
from jax import lax
from jax.experimental import pallas as pl
from jax.experimental.pallas import tpu as pltpu
from jax.experimental.pallas import tpu_sc as plsc

W = 2
NSIDE = 262
NN = NSIDE * NSIDE
K = (2 * W + 1) ** 2
H = 128
D = 128
B = 16384

NC, NS, L = 2, 16, 16
NW = NC * NS
BPW = B // NW

OFFS = [dx * NSIDE + dy for dx in range(-W, W + 1) for dy in range(-W, W + 1)]
OFF_MIN = -OFFS[0]
VIEW_LEN = NN - 2 * OFF_MIN

SPAN = 2176
NWIN = SPAN // 128
OUT_ROWS = NW * SPAN
MAXKEY = 0x7FFFFFFF

def _mesh():
    return plsc.VectorSubcoreMesh(core_axis_name="c", subcore_axis_name="s")


def _worker_id():
    return lax.axis_index("s") * NC + lax.axis_index("c")


def _sc_gather_body(gx_hbm, gy_hbm, mem2_hbm, out_hbm, gxv, gyv, idx2, rows, s0, s1):
    wid = _worker_id()
    base = wid * BPW
    pltpu.sync_copy(gx_hbm.at[pl.ds(base, BPW)], gxv)
    pltpu.sync_copy(gy_hbm.at[pl.ds(base, BPW)], gyv)

    def idx_body(g, _):
        s = pl.ds(g * L, L)
        f = gxv[s] * NSIDE + gyv[s]
        for k in range(K):
            idx2[pl.ds(k * BPW + g * L, L)] = f + OFFS[k]
        return 0

    lax.fori_loop(0, BPW // L, idx_body, 0)

    sems = [s0, s1]
    nt = K * (BPW // 128)

    def issue(t):
        k, w = divmod(t, BPW // 128)
        idxrow = idx2.at[pl.ds(k * BPW + w * 128, 128)]
        return pltpu.async_copy(mem2_hbm.at[idxrow], rows.at[t % 2], sems[t % 2])

    pending = issue(0)
    for t in range(nt):
        pending.wait()
        nxt = issue(t + 1) if t + 1 < nt else None
        k, w = divmod(t, BPW // 128)
        pltpu.sync_copy(rows.at[t % 2], out_hbm.at[k, pl.ds(base + w * 128, 128), :])
        pending = nxt


def _sc_gather(gx, gy, mem2):
    fn = pl.kernel(
        _sc_gather_body,
        mesh=_mesh(),
        out_type=jax.ShapeDtypeStruct((K, B, H), jnp.float32),
        scratch_types=[
            pltpu.VMEM((BPW,), jnp.int32),
            pltpu.VMEM((BPW,), jnp.int32),
            pltpu.VMEM((K * BPW,), jnp.int32),
            pltpu.VMEM((2, 128, H), jnp.float32),
            pltpu.SemaphoreType.DMA,
            pltpu.SemaphoreType.DMA,
        ],
    )
    return fn(gx, gy, mem2)


BB = 512


def _tc_body(x_ref, hx_ref, wih_ref, whh_ref, bih_ref, bhh_ref, cs_ref,
             hyy_ref, upd_ref):
    xg = jnp.dot(x_ref[...], wih_ref[...],
                 preferred_element_type=jnp.float32) + bih_ref[...]
    hg = jnp.dot(hx_ref[...], whh_ref[...],
                 preferred_element_type=jnp.float32) + bhh_ref[...]
    r = jax.nn.sigmoid(xg[:, :H] + hg[:, :H])
    u = jax.nn.sigmoid(xg[:, H:2 * H] + hg[:, H:2 * H])
    sg = jax.nn.sigmoid(xg[:, 3 * H:] + hg[:, 3 * H:])
    ng = jnp.tanh(xg[:, 2 * H:3 * H] + r * hg[:, 2 * H:3 * H])

    scores = jnp.concatenate(
        [jnp.sum(cs_ref[k] * ng, axis=1, keepdims=True) for k in range(K)],
        axis=1)
    m = jnp.max(scores, axis=1, keepdims=True)
    e = jnp.exp(scores - m)
    inv = 1.0 / jnp.sum(e, axis=1, keepdims=True)
    acc = jnp.zeros((BB, H), jnp.float32)
    for k in range(K):
        acc = acc + (e[:, k:k + 1] * inv) * cs_ref[k]
    curr = ng + sg * acc
    hyy = curr + u * (hx_ref[...] - curr)
    hyy_ref[...] = hyy
    upd_ref[...] = sg * cs_ref[W * (2 * W + 1) + W] + (1.0 - sg) * hyy


def _tc_main(x, hx, wih_t, whh_t, bih, bhh, cs):
    grid = (B // BB,)
    return pl.pallas_call(
        _tc_body,
        grid=grid,
        in_specs=[
            pl.BlockSpec((BB, D), lambda i: (i, 0)),
            pl.BlockSpec((BB, H), lambda i: (i, 0)),
            pl.BlockSpec((D, 4 * H), lambda i: (0, 0)),
            pl.BlockSpec((H, 4 * H), lambda i: (0, 0)),
            pl.BlockSpec((1, 4 * H), lambda i: (0, 0)),
            pl.BlockSpec((1, 4 * H), lambda i: (0, 0)),
            pl.BlockSpec((K, BB, H), lambda i: (0, i, 0)),
        ],
        out_specs=[
            pl.BlockSpec((BB, H), lambda i: (i, 0)),
            pl.BlockSpec((BB, H), lambda i: (i, 0)),
        ],
        out_shape=[
            jax.ShapeDtypeStruct((B, H), jnp.float32),
            jax.ShapeDtypeStruct((B, H), jnp.float32),
        ],
    )(x, hx, wih_t, whh_t, bih, bhh, cs)


def _sc_winner_body(gx_hbm, gy_hbm, lw_hbm, gxv, gyv, lwin):
    wid = _worker_id()
    base = wid * BPW
    pltpu.sync_copy(gx_hbm.at[pl.ds(base, BPW)], gxv)
    pltpu.sync_copy(gy_hbm.at[pl.ds(base, BPW)], gyv)

    iota = jnp.arange(L, dtype=jnp.int32)
    neg1 = jnp.full((L,), -1, jnp.int32)

    def init_body(j, _):
        lwin[pl.ds(j * L, L)] = neg1
        return 0

    lax.fori_loop(0, OUT_ROWS // L, init_body, 0)

    def group_body(g, _):
        s = pl.ds(g * L, L)
        cell16 = gxv[s] * NSIDE + gyv[s]
        for j in range(L):
            cell = cell16[j]
            wbase = pl.multiple_of((cell >> 4) << 4, L)
            lane = cell & (L - 1)
            win = lwin[pl.ds(wbase, L)]
            lwin[pl.ds(wbase, L)] = jnp.where(iota == lane, base + g * L + j, win)
        return 0

    lax.fori_loop(0, BPW // L, group_body, 0)
    pltpu.sync_copy(lwin, lw_hbm.at[wid])


def _sc_winner(gx, gy):
    fn = pl.kernel(
        _sc_winner_body,
        mesh=_mesh(),
        out_type=jax.ShapeDtypeStruct((NW, OUT_ROWS), jnp.int32),
        scratch_types=[
            pltpu.VMEM((BPW,), jnp.int32),
            pltpu.VMEM((BPW,), jnp.int32),
            pltpu.VMEM((OUT_ROWS,), jnp.int32),
        ],
    )
    return fn(gx, gy)


def _sc_scatter_body(lw_hbm, comb_hbm, out_hbm, acc, buf, idxw, rows, sem):
    wid = _worker_id()
    lo = wid * SPAN

    pltpu.sync_copy(lw_hbm.at[0, pl.ds(lo, SPAN)], acc)

    def merge_t(t, _):
        pltpu.sync_copy(lw_hbm.at[t, pl.ds(lo, SPAN)], buf)

        def merge_j(j, _):
            s = pl.ds(j * L, L)
            acc[s] = jnp.maximum(acc[s], buf[s])
            return 0

        lax.fori_loop(0, SPAN // L, merge_j, 0)
        return 0

    lax.fori_loop(1, NW, merge_t, 0)

    iota = jnp.arange(L, dtype=jnp.int32)

    def win_body(w, _):
        for g in range(128 // L):
            c16 = lo + w * 128 + g * L + iota
            wv = acc[pl.ds(w * 128 + g * L, L)]
            src = jnp.where(wv >= 0, NN + wv, jnp.minimum(c16, NN - 1))
            idxw[pl.ds(g * L, L)] = src
        pltpu.async_copy(comb_hbm.at[idxw], rows, sem).wait()
        pltpu.sync_copy(rows, out_hbm.at[pl.ds(lo + w * 128, 128), :])
        return 0

    lax.fori_loop(0, NWIN, win_body, 0)


def _sc_scatter(lw, combined):
    fn = pl.kernel(
        _sc_scatter_body,
        mesh=_mesh(),
        out_type=jax.ShapeDtypeStruct((OUT_ROWS, H), jnp.float32),
        scratch_types=[
            pltpu.VMEM((SPAN,), jnp.int32),
            pltpu.VMEM((SPAN,), jnp.int32),
            pltpu.VMEM((128,), jnp.int32),
            pltpu.VMEM((128, H), jnp.float32),
            pltpu.SemaphoreType.DMA,
        ],
    )
    return fn(lw, combined)


def kernel(input, hx, weight_ih, weight_hh, bias_ih, bias_hh, memory):
    x = input[:, :D]
    coords = input[:, D:].astype(jnp.int32) + W
    gx = coords[:, 0]
    gy = coords[:, 1]
    mem2 = memory.reshape(NN, H)

    cs = _sc_gather(gx, gy, mem2)

    hyy, updates = _tc_main(
        x, hx, weight_ih.T, weight_hh.T,
        bias_ih.reshape(1, 4 * H), bias_hh.reshape(1, 4 * H), cs)

    combined = jnp.concatenate([mem2, updates], axis=0)
    lw = _sc_winner(gx, gy)
    outp = _sc_scatter(lw, combined)
    new_mem = outp[:NN].reshape(NSIDE, NSIDE, H)
    return hyy, new_mem


# speedup vs baseline: 14.7051x; 1.1589x over previous
import functools

import jax
import jax.numpy as jnp
from jax import lax
from jax.experimental import pallas as pl
from jax.experimental.pallas import tpu as pltpu
from jax.experimental.pallas import tpu_sc as plsc

W = 2
NSIDE = 262
NN = NSIDE * NSIDE
K = (2 * W + 1) ** 2
H = 128
D = 128
B = 16384

NC, NS, L = 2, 16, 16
NW = NC * NS
BPW = B // NW

OFFS = [dx * NSIDE + dy for dx in range(-W, W + 1) for dy in range(-W, W + 1)]
OFF_MIN = -OFFS[0]
VIEW_LEN = NN - 2 * OFF_MIN

SPAN = 2176
NWIN = SPAN // 128
OUT_ROWS = NW * SPAN
MAXKEY = 0x7FFFFFFF

def _mesh():
    return plsc.VectorSubcoreMesh(core_axis_name="c", subcore_axis_name="s")


def _worker_id():
    return lax.axis_index("s") * NC + lax.axis_index("c")


GWIN = 64
GNBUF = 8
GDEPTH = 4


def _sc_gather_body(gx_hbm, gy_hbm, mem2_hbm, out_hbm, gxv, gyv, idx2, rows,
                    gsems, osems):
    wid = _worker_id()
    base = wid * BPW
    pltpu.sync_copy(gx_hbm.at[pl.ds(base, BPW)], gxv)
    pltpu.sync_copy(gy_hbm.at[pl.ds(base, BPW)], gyv)

    def idx_body(g, _):
        s = pl.ds(g * L, L)
        f = gxv[s] * NSIDE + gyv[s]
        for k in range(K):
            idx2[pl.ds(k * BPW + g * L, L)] = f + OFFS[k]
        return 0

    lax.fori_loop(0, BPW // L, idx_body, 0)

    wpk = BPW // GWIN
    nt = K * wpk

    def gissue(t):
        k, w = divmod(t, wpk)
        idxrow = idx2.at[pl.ds(k * BPW + w * GWIN, GWIN)]
        return pltpu.async_copy(mem2_hbm.at[idxrow], rows.at[t % GNBUF],
                                gsems[t % GNBUF])

    def oissue(t):
        k, w = divmod(t, wpk)
        return pltpu.async_copy(rows.at[t % GNBUF],
                                out_hbm.at[k, pl.ds(base + w * GWIN, GWIN), :],
                                osems[t % GNBUF])

    gh = {t: gissue(t) for t in range(GDEPTH)}
    oh = {}
    for t in range(nt):
        gh.pop(t).wait()
        oh[t] = oissue(t)
        u = t + GDEPTH
        if u < nt:
            if u >= GNBUF:
                oh.pop(u - GNBUF).wait()
            gh[u] = gissue(u)
    for t in sorted(oh):
        oh[t].wait()


def _sc_gather(gx, gy, mem2):
    fn = pl.kernel(
        _sc_gather_body,
        mesh=_mesh(),
        out_type=jax.ShapeDtypeStruct((K, B, H), jnp.float32),
        scratch_types=[
            pltpu.VMEM((BPW,), jnp.int32),
            pltpu.VMEM((BPW,), jnp.int32),
            pltpu.VMEM((K * BPW,), jnp.int32),
            pltpu.VMEM((GNBUF, GWIN, H), jnp.float32),
            [pltpu.SemaphoreType.DMA] * GNBUF,
            [pltpu.SemaphoreType.DMA] * GNBUF,
        ],
    )
    return fn(gx, gy, mem2)


BB = 512


def _tc_body(x_ref, hx_ref, wih_ref, whh_ref, bih_ref, bhh_ref, cs_ref,
             hyy_ref, upd_ref):
    xg = jnp.dot(x_ref[...], wih_ref[...],
                 preferred_element_type=jnp.float32) + bih_ref[...]
    hg = jnp.dot(hx_ref[...], whh_ref[...],
                 preferred_element_type=jnp.float32) + bhh_ref[...]
    r = jax.nn.sigmoid(xg[:, :H] + hg[:, :H])
    u = jax.nn.sigmoid(xg[:, H:2 * H] + hg[:, H:2 * H])
    sg = jax.nn.sigmoid(xg[:, 3 * H:] + hg[:, 3 * H:])
    ng = jnp.tanh(xg[:, 2 * H:3 * H] + r * hg[:, 2 * H:3 * H])

    scores = jnp.concatenate(
        [jnp.sum(cs_ref[k] * ng, axis=1, keepdims=True) for k in range(K)],
        axis=1)
    m = jnp.max(scores, axis=1, keepdims=True)
    e = jnp.exp(scores - m)
    inv = 1.0 / jnp.sum(e, axis=1, keepdims=True)
    acc = jnp.zeros((BB, H), jnp.float32)
    for k in range(K):
        acc = acc + (e[:, k:k + 1] * inv) * cs_ref[k]
    curr = ng + sg * acc
    hyy = curr + u * (hx_ref[...] - curr)
    hyy_ref[...] = hyy
    upd_ref[...] = sg * cs_ref[W * (2 * W + 1) + W] + (1.0 - sg) * hyy


def _tc_main(x, hx, wih_t, whh_t, bih, bhh, cs):
    grid = (B // BB,)
    return pl.pallas_call(
        _tc_body,
        grid=grid,
        in_specs=[
            pl.BlockSpec((BB, D), lambda i: (i, 0)),
            pl.BlockSpec((BB, H), lambda i: (i, 0)),
            pl.BlockSpec((D, 4 * H), lambda i: (0, 0)),
            pl.BlockSpec((H, 4 * H), lambda i: (0, 0)),
            pl.BlockSpec((1, 4 * H), lambda i: (0, 0)),
            pl.BlockSpec((1, 4 * H), lambda i: (0, 0)),
            pl.BlockSpec((K, BB, H), lambda i: (0, i, 0)),
        ],
        out_specs=[
            pl.BlockSpec((BB, H), lambda i: (i, 0)),
            pl.BlockSpec((BB, H), lambda i: (i, 0)),
        ],
        out_shape=[
            jax.ShapeDtypeStruct((B, H), jnp.float32),
            jax.ShapeDtypeStruct((B, H), jnp.float32),
        ],
    )(x, hx, wih_t, whh_t, bih, bhh, cs)


def _sc_winner_body(gx_hbm, gy_hbm, lw_hbm, gxv, gyv, lwin):
    wid = _worker_id()
    base = wid * BPW
    pltpu.sync_copy(gx_hbm.at[pl.ds(base, BPW)], gxv)
    pltpu.sync_copy(gy_hbm.at[pl.ds(base, BPW)], gyv)

    iota = jnp.arange(L, dtype=jnp.int32)
    neg1 = jnp.full((L,), -1, jnp.int32)

    def init_body(j, _):
        lwin[pl.ds(j * L, L)] = neg1
        return 0

    lax.fori_loop(0, OUT_ROWS // L, init_body, 0)

    def group_body(g, _):
        s = pl.ds(g * L, L)
        cell16 = gxv[s] * NSIDE + gyv[s]
        for j in range(L):
            cell = cell16[j]
            wbase = pl.multiple_of((cell >> 4) << 4, L)
            lane = cell & (L - 1)
            win = lwin[pl.ds(wbase, L)]
            lwin[pl.ds(wbase, L)] = jnp.where(iota == lane, base + g * L + j, win)
        return 0

    lax.fori_loop(0, BPW // L, group_body, 0)
    pltpu.sync_copy(lwin, lw_hbm.at[wid])


def _sc_winner(gx, gy):
    fn = pl.kernel(
        _sc_winner_body,
        mesh=_mesh(),
        out_type=jax.ShapeDtypeStruct((NW, OUT_ROWS), jnp.int32),
        scratch_types=[
            pltpu.VMEM((BPW,), jnp.int32),
            pltpu.VMEM((BPW,), jnp.int32),
            pltpu.VMEM((OUT_ROWS,), jnp.int32),
        ],
    )
    return fn(gx, gy)


SNBUF = 4
SDEPTH = 2


def _sc_scatter_body(lw_hbm, comb_hbm, out_hbm, allw, allw2, acc, idxw, rows,
                     gsems, osems):
    wid = _worker_id()
    lo = wid * SPAN

    QH = 8
    mbufs = [allw, allw2]

    def mload(q):
        return pltpu.async_copy(lw_hbm.at[pl.ds(q * QH, QH), pl.ds(lo, SPAN)],
                                mbufs[q % 2], gsems[q % 2])

    mh = mload(0)
    for q in range(NW // QH):
        mh.wait()
        if q + 1 < NW // QH:
            mh = mload(q + 1)
        mb = mbufs[q % 2]

        def merge_j(j, _, mb=mb, first=(q == 0)):
            s = pl.ds(j * L, L)
            m = mb[0, s]
            for t in range(1, QH):
                m = jnp.maximum(m, mb[t, s])
            acc[s] = m if first else jnp.maximum(acc[s], m)
            return 0

        lax.fori_loop(0, SPAN // L, merge_j, 0)

    iota = jnp.arange(L, dtype=jnp.int32)

    def build_idx(w):
        for g in range(128 // L):
            c16 = lo + w * 128 + g * L + iota
            wv = acc[pl.ds(w * 128 + g * L, L)]
            src = jnp.where(wv >= 0, NN + wv, jnp.minimum(c16, NN - 1))
            idxw[pl.ds((w % SNBUF) * 128 + g * L, L)] = src

    def gissue(w):
        idxrow = idxw.at[pl.ds((w % SNBUF) * 128, 128)]
        return pltpu.async_copy(comb_hbm.at[idxrow], rows.at[w % SNBUF],
                                gsems[w % SNBUF])

    def oissue(w):
        return pltpu.async_copy(rows.at[w % SNBUF],
                                out_hbm.at[pl.ds(lo + w * 128, 128), :],
                                osems[w % SNBUF])

    gh, oh = {}, {}
    for w in range(SDEPTH):
        build_idx(w)
        gh[w] = gissue(w)
    for w in range(NWIN):
        gh.pop(w).wait()
        oh[w] = oissue(w)
        u = w + SDEPTH
        if u < NWIN:
            if u >= SNBUF:
                oh.pop(u - SNBUF).wait()
            build_idx(u)
            gh[u] = gissue(u)
    for w in sorted(oh):
        oh[w].wait()


def _sc_scatter(lw, combined):
    fn = pl.kernel(
        _sc_scatter_body,
        mesh=_mesh(),
        out_type=jax.ShapeDtypeStruct((OUT_ROWS, H), jnp.float32),
        scratch_types=[
            pltpu.VMEM((8, SPAN), jnp.int32),
            pltpu.VMEM((8, SPAN), jnp.int32),
            pltpu.VMEM((SPAN,), jnp.int32),
            pltpu.VMEM((SNBUF * 128,), jnp.int32),
            pltpu.VMEM((SNBUF, 128, H), jnp.float32),
            [pltpu.SemaphoreType.DMA] * SNBUF,
            [pltpu.SemaphoreType.DMA] * SNBUF,
        ],
    )
    return fn(lw, combined)


def kernel(input, hx, weight_ih, weight_hh, bias_ih, bias_hh, memory):
    x = input[:, :D]
    coords = input[:, D:].astype(jnp.int32) + W
    gx = coords[:, 0]
    gy = coords[:, 1]
    mem2 = memory.reshape(NN, H)

    cs = _sc_gather(gx, gy, mem2)

    hyy, updates = _tc_main(
        x, hx, weight_ih.T, weight_hh.T,
        bias_ih.reshape(1, 4 * H), bias_hh.reshape(1, 4 * H), cs)

    combined = jnp.concatenate([mem2, updates], axis=0)
    lw = _sc_winner(gx, gy)
    outp = _sc_scatter(lw, combined)
    new_mem = outp[:NN].reshape(NSIDE, NSIDE, H)
    return hyy, new_mem


# speedup vs baseline: 16.6609x; 1.1330x over previous
import functools

import numpy as _np

import jax
import jax.numpy as jnp
from jax import lax
from jax.experimental import pallas as pl
from jax.experimental.pallas import tpu as pltpu
from jax.experimental.pallas import tpu_sc as plsc

W = 2
NSIDE = 262
NN = NSIDE * NSIDE
K = (2 * W + 1) ** 2
H = 128
D = 128
B = 16384

NC, NS, L = 2, 16, 16
NW = NC * NS
BPW = B // NW

OFFS = [dx * NSIDE + dy for dx in range(-W, W + 1) for dy in range(-W, W + 1)]
OFF_MIN = -OFFS[0]
VIEW_LEN = NN - 2 * OFF_MIN

SPAN = 2176
NWIN = SPAN // 128
OUT_ROWS = NW * SPAN
MAXKEY = 0x7FFFFFFF

def _mesh():
    return plsc.VectorSubcoreMesh(core_axis_name="c", subcore_axis_name="s")


def _worker_id():
    return lax.axis_index("s") * NC + lax.axis_index("c")


GWIN = 64
GNBUF = 8
GDEPTH = 4


def _sc_gather_body(gx_hbm, gy_hbm, mem2_hbm, out_hbm, gxv, gyv, idx2, rows,
                    gsems, osems):
    wid = _worker_id()
    base = wid * BPW
    pltpu.sync_copy(gx_hbm.at[pl.ds(base, BPW)], gxv)
    pltpu.sync_copy(gy_hbm.at[pl.ds(base, BPW)], gyv)

    def idx_body(g, _):
        s = pl.ds(g * L, L)
        f = gxv[s] * NSIDE + gyv[s]
        for k in range(K):
            idx2[pl.ds(k * BPW + g * L, L)] = f + OFFS[k]
        return 0

    lax.fori_loop(0, BPW // L, idx_body, 0)

    wpk = BPW // GWIN
    nt = K * wpk

    def gissue(t):
        k, w = divmod(t, wpk)
        idxrow = idx2.at[pl.ds(k * BPW + w * GWIN, GWIN)]
        return pltpu.async_copy(mem2_hbm.at[idxrow], rows.at[t % GNBUF],
                                gsems[t % GNBUF])

    def oissue(t):
        k, w = divmod(t, wpk)
        return pltpu.async_copy(rows.at[t % GNBUF],
                                out_hbm.at[k, pl.ds(base + w * GWIN, GWIN), :],
                                osems[t % GNBUF])

    gh = {t: gissue(t) for t in range(GDEPTH)}
    oh = {}
    for t in range(nt):
        gh.pop(t).wait()
        oh[t] = oissue(t)
        u = t + GDEPTH
        if u < nt:
            if u >= GNBUF:
                oh.pop(u - GNBUF).wait()
            gh[u] = gissue(u)
    for t in sorted(oh):
        oh[t].wait()


def _sc_gather(gx, gy, mem2):
    fn = pl.kernel(
        _sc_gather_body,
        mesh=_mesh(),
        out_type=jax.ShapeDtypeStruct((K, B, H), jnp.float32),
        scratch_types=[
            pltpu.VMEM((BPW,), jnp.int32),
            pltpu.VMEM((BPW,), jnp.int32),
            pltpu.VMEM((K * BPW,), jnp.int32),
            pltpu.VMEM((GNBUF, GWIN, H), jnp.float32),
            [pltpu.SemaphoreType.DMA] * GNBUF,
            [pltpu.SemaphoreType.DMA] * GNBUF,
        ],
    )
    return fn(gx, gy, mem2)


BB = 1024


def _tc_body(x_ref, hx_ref, wih_ref, whh_ref, bih_ref, bhh_ref, cs_ref,
             sel_ref, hyy_ref, upd_ref):
    xg = jnp.dot(x_ref[...], wih_ref[...],
                 preferred_element_type=jnp.float32) + bih_ref[...]
    hg = jnp.dot(hx_ref[...], whh_ref[...],
                 preferred_element_type=jnp.float32) + bhh_ref[...]
    r = jax.nn.sigmoid(xg[:, :H] + hg[:, :H])
    u = jax.nn.sigmoid(xg[:, H:2 * H] + hg[:, H:2 * H])
    sg = jax.nn.sigmoid(xg[:, 3 * H:] + hg[:, 3 * H:])
    ng = jnp.tanh(xg[:, 2 * H:3 * H] + r * hg[:, 2 * H:3 * H])

    scores = jnp.concatenate(
        [jnp.sum(cs_ref[k] * ng, axis=1, keepdims=True) for k in range(K)],
        axis=1)
    m = jnp.max(scores, axis=1, keepdims=True)
    e = jnp.exp(scores - m)
    inv = 1.0 / jnp.sum(e, axis=1, keepdims=True)
    attn = jnp.pad(e * inv, ((0, 0), (0, H - K)))
    acc = jnp.zeros((BB, H), jnp.float32)
    for k in range(K):
        ak = jnp.dot(attn, sel_ref[k], preferred_element_type=jnp.float32)
        acc = acc + ak * cs_ref[k]
    curr = ng + sg * acc
    hyy = curr + u * (hx_ref[...] - curr)
    hyy_ref[...] = hyy
    upd_ref[...] = sg * cs_ref[W * (2 * W + 1) + W] + (1.0 - sg) * hyy


def _tc_main(x, hx, wih_t, whh_t, bih, bhh, cs):
    grid = (B // BB,)
    sel = jnp.asarray(
        (_np.arange(H)[None, :, None] == _np.arange(K)[:, None, None])
        .astype(_np.float32) * _np.ones((1, 1, H), _np.float32))
    return pl.pallas_call(
        _tc_body,
        grid=grid,
        in_specs=[
            pl.BlockSpec((BB, D), lambda i: (i, 0)),
            pl.BlockSpec((BB, H), lambda i: (i, 0)),
            pl.BlockSpec((D, 4 * H), lambda i: (0, 0)),
            pl.BlockSpec((H, 4 * H), lambda i: (0, 0)),
            pl.BlockSpec((1, 4 * H), lambda i: (0, 0)),
            pl.BlockSpec((1, 4 * H), lambda i: (0, 0)),
            pl.BlockSpec((K, BB, H), lambda i: (0, i, 0)),
            pl.BlockSpec((K, H, H), lambda i: (0, 0, 0)),
        ],
        out_specs=[
            pl.BlockSpec((BB, H), lambda i: (i, 0)),
            pl.BlockSpec((BB, H), lambda i: (i, 0)),
        ],
        out_shape=[
            jax.ShapeDtypeStruct((B, H), jnp.float32),
            jax.ShapeDtypeStruct((B, H), jnp.float32),
        ],
    )(x, hx, wih_t, whh_t, bih, bhh, cs, sel)


def _sc_winner_body(gx_hbm, gy_hbm, lw_hbm, gxv, gyv, lwin):
    wid = _worker_id()
    base = wid * BPW
    pltpu.sync_copy(gx_hbm.at[pl.ds(base, BPW)], gxv)
    pltpu.sync_copy(gy_hbm.at[pl.ds(base, BPW)], gyv)

    iota = jnp.arange(L, dtype=jnp.int32)
    neg1 = jnp.full((L,), -1, jnp.int32)

    def init_body(j, _):
        lwin[pl.ds(j * L, L)] = neg1
        return 0

    lax.fori_loop(0, OUT_ROWS // L, init_body, 0)

    def group_body(g, _):
        s = pl.ds(g * L, L)
        cell16 = gxv[s] * NSIDE + gyv[s]
        for j in range(L):
            cell = cell16[j]
            wbase = pl.multiple_of((cell >> 4) << 4, L)
            lane = cell & (L - 1)
            win = lwin[pl.ds(wbase, L)]
            lwin[pl.ds(wbase, L)] = jnp.where(iota == lane, base + g * L + j, win)
        return 0

    lax.fori_loop(0, BPW // L, group_body, 0)
    pltpu.sync_copy(lwin, lw_hbm.at[wid])


def _sc_winner(gx, gy):
    fn = pl.kernel(
        _sc_winner_body,
        mesh=_mesh(),
        out_type=jax.ShapeDtypeStruct((NW, OUT_ROWS), jnp.int32),
        scratch_types=[
            pltpu.VMEM((BPW,), jnp.int32),
            pltpu.VMEM((BPW,), jnp.int32),
            pltpu.VMEM((OUT_ROWS,), jnp.int32),
        ],
    )
    return fn(gx, gy)


SNBUF = 4
SDEPTH = 2


def _sc_scatter_body(lw_hbm, comb_hbm, out_hbm, allw, allw2, acc, idxw, rows,
                     gsems, osems):
    wid = _worker_id()
    lo = wid * SPAN

    QH = 8
    mbufs = [allw, allw2]

    def mload(q):
        return pltpu.async_copy(lw_hbm.at[pl.ds(q * QH, QH), pl.ds(lo, SPAN)],
                                mbufs[q % 2], gsems[q % 2])

    mh = mload(0)
    for q in range(NW // QH):
        mh.wait()
        if q + 1 < NW // QH:
            mh = mload(q + 1)
        mb = mbufs[q % 2]

        def merge_j(j, _, mb=mb, first=(q == 0)):
            s = pl.ds(j * L, L)
            m = mb[0, s]
            for t in range(1, QH):
                m = jnp.maximum(m, mb[t, s])
            acc[s] = m if first else jnp.maximum(acc[s], m)
            return 0

        lax.fori_loop(0, SPAN // L, merge_j, 0)

    iota = jnp.arange(L, dtype=jnp.int32)

    def build_idx(w):
        for g in range(128 // L):
            c16 = lo + w * 128 + g * L + iota
            wv = acc[pl.ds(w * 128 + g * L, L)]
            src = jnp.where(wv >= 0, NN + wv, jnp.minimum(c16, NN - 1))
            idxw[pl.ds((w % SNBUF) * 128 + g * L, L)] = src

    def gissue(w):
        idxrow = idxw.at[pl.ds((w % SNBUF) * 128, 128)]
        return pltpu.async_copy(comb_hbm.at[idxrow], rows.at[w % SNBUF],
                                gsems[w % SNBUF])

    def oissue(w):
        return pltpu.async_copy(rows.at[w % SNBUF],
                                out_hbm.at[pl.ds(lo + w * 128, 128), :],
                                osems[w % SNBUF])

    gh, oh = {}, {}
    for w in range(SDEPTH):
        build_idx(w)
        gh[w] = gissue(w)
    for w in range(NWIN):
        gh.pop(w).wait()
        oh[w] = oissue(w)
        u = w + SDEPTH
        if u < NWIN:
            if u >= SNBUF:
                oh.pop(u - SNBUF).wait()
            build_idx(u)
            gh[u] = gissue(u)
    for w in sorted(oh):
        oh[w].wait()


def _sc_scatter(lw, combined):
    fn = pl.kernel(
        _sc_scatter_body,
        mesh=_mesh(),
        out_type=jax.ShapeDtypeStruct((OUT_ROWS, H), jnp.float32),
        scratch_types=[
            pltpu.VMEM((8, SPAN), jnp.int32),
            pltpu.VMEM((8, SPAN), jnp.int32),
            pltpu.VMEM((SPAN,), jnp.int32),
            pltpu.VMEM((SNBUF * 128,), jnp.int32),
            pltpu.VMEM((SNBUF, 128, H), jnp.float32),
            [pltpu.SemaphoreType.DMA] * SNBUF,
            [pltpu.SemaphoreType.DMA] * SNBUF,
        ],
    )
    return fn(lw, combined)


def kernel(input, hx, weight_ih, weight_hh, bias_ih, bias_hh, memory):
    x = input[:, :D]
    coords = input[:, D:].astype(jnp.int32) + W
    gx = coords[:, 0]
    gy = coords[:, 1]
    mem2 = memory.reshape(NN, H)

    cs = _sc_gather(gx, gy, mem2)

    hyy, updates = _tc_main(
        x, hx, weight_ih.T, weight_hh.T,
        bias_ih.reshape(1, 4 * H), bias_hh.reshape(1, 4 * H), cs)

    combined = jnp.concatenate([mem2, updates], axis=0)
    lw = _sc_winner(gx, gy)
    outp = _sc_scatter(lw, combined)
    new_mem = outp[:NN].reshape(NSIDE, NSIDE, H)
    return hyy, new_mem


# speedup vs baseline: 16.7336x; 1.0044x over previous
import functools

import numpy as _np

import jax
import jax.numpy as jnp
from jax import lax
from jax.experimental import pallas as pl
from jax.experimental.pallas import tpu as pltpu
from jax.experimental.pallas import tpu_sc as plsc

W = 2
NSIDE = 262
NN = NSIDE * NSIDE
K = (2 * W + 1) ** 2
H = 128
D = 128
B = 16384

NC, NS, L = 2, 16, 16
NW = NC * NS
BPW = B // NW

OFFS = [dx * NSIDE + dy for dx in range(-W, W + 1) for dy in range(-W, W + 1)]
OFF_MIN = -OFFS[0]
VIEW_LEN = NN - 2 * OFF_MIN

SPAN = 2176
NWIN = SPAN // 128
OUT_ROWS = NW * SPAN
NN_ALIGN = (NN // 128) * 128
MAXKEY = 0x7FFFFFFF

def _mesh():
    return plsc.VectorSubcoreMesh(core_axis_name="c", subcore_axis_name="s")


def _worker_id():
    return lax.axis_index("s") * NC + lax.axis_index("c")


GWIN = 128
GNBUF = 6
GDEPTH = 4


def _sc_gather_body(gx_hbm, gy_hbm, mem2_hbm, out_hbm, gxv, gyv, idx2, rows,
                    gsems, osems):
    wid = _worker_id()
    base = wid * BPW
    pltpu.sync_copy(gx_hbm.at[pl.ds(base, BPW)], gxv)
    pltpu.sync_copy(gy_hbm.at[pl.ds(base, BPW)], gyv)

    def idx_body(g, _):
        s = pl.ds(g * L, L)
        f = gxv[s] * NSIDE + gyv[s]
        for k in range(K):
            idx2[pl.ds(k * BPW + g * L, L)] = f + OFFS[k]
        return 0

    lax.fori_loop(0, BPW // L, idx_body, 0)

    wpk = BPW // GWIN
    nt = K * wpk

    def gissue(t):
        k, w = divmod(t, wpk)
        idxrow = idx2.at[pl.ds(k * BPW + w * GWIN, GWIN)]
        return pltpu.async_copy(mem2_hbm.at[idxrow], rows.at[t % GNBUF],
                                gsems[t % GNBUF])

    def oissue(t):
        k, w = divmod(t, wpk)
        return pltpu.async_copy(rows.at[t % GNBUF],
                                out_hbm.at[k, pl.ds(base + w * GWIN, GWIN), :],
                                osems[t % GNBUF])

    gh = {t: gissue(t) for t in range(GDEPTH)}
    oh = {}
    for t in range(nt):
        gh.pop(t).wait()
        oh[t] = oissue(t)
        u = t + GDEPTH
        if u < nt:
            if u >= GNBUF:
                oh.pop(u - GNBUF).wait()
            gh[u] = gissue(u)
    for t in sorted(oh):
        oh[t].wait()


def _sc_gather(gx, gy, mem2):
    fn = pl.kernel(
        _sc_gather_body,
        mesh=_mesh(),
        out_type=jax.ShapeDtypeStruct((K, B, H), jnp.float32),
        scratch_types=[
            pltpu.VMEM((BPW,), jnp.int32),
            pltpu.VMEM((BPW,), jnp.int32),
            pltpu.VMEM((K * BPW,), jnp.int32),
            pltpu.VMEM((GNBUF, GWIN, H), jnp.float32),
            [pltpu.SemaphoreType.DMA] * GNBUF,
            [pltpu.SemaphoreType.DMA] * GNBUF,
        ],
    )
    return fn(gx, gy, mem2)


BB = 1024


def _tc_body(x_ref, hx_ref, wih_ref, whh_ref, bih_ref, bhh_ref, cs_ref,
             sel_ref, hyy_ref, upd_ref):
    xg = jnp.dot(x_ref[:, :D], wih_ref[...],
                 preferred_element_type=jnp.float32) + bih_ref[...]
    hg = jnp.dot(hx_ref[...], whh_ref[...],
                 preferred_element_type=jnp.float32) + bhh_ref[...]
    r = jax.nn.sigmoid(xg[:, :H] + hg[:, :H])
    u = jax.nn.sigmoid(xg[:, H:2 * H] + hg[:, H:2 * H])
    sg = jax.nn.sigmoid(xg[:, 3 * H:] + hg[:, 3 * H:])
    ng = jnp.tanh(xg[:, 2 * H:3 * H] + r * hg[:, 2 * H:3 * H])

    scores = jnp.concatenate(
        [jnp.sum(cs_ref[k] * ng, axis=1, keepdims=True) for k in range(K)],
        axis=1)
    m = jnp.max(scores, axis=1, keepdims=True)
    e = jnp.exp(scores - m)
    inv = 1.0 / jnp.sum(e, axis=1, keepdims=True)
    attn = jnp.pad(e * inv, ((0, 0), (0, H - K)))
    acc = jnp.zeros((BB, H), jnp.float32)
    for k in range(K):
        ak = jnp.dot(attn, sel_ref[k], preferred_element_type=jnp.float32)
        acc = acc + ak * cs_ref[k]
    curr = ng + sg * acc
    hyy = curr + u * (hx_ref[...] - curr)
    hyy_ref[...] = hyy
    upd_ref[...] = sg * cs_ref[W * (2 * W + 1) + W] + (1.0 - sg) * hyy


def _tc_main(x, hx, wih_t, whh_t, bih, bhh, cs):
    grid = (B // BB,)
    sel = jnp.asarray(
        (_np.arange(H)[None, :, None] == _np.arange(K)[:, None, None])
        .astype(_np.float32) * _np.ones((1, 1, H), _np.float32))
    return pl.pallas_call(
        _tc_body,
        grid=grid,
        in_specs=[
            pl.BlockSpec((BB, D + 2), lambda i: (i, 0)),
            pl.BlockSpec((BB, H), lambda i: (i, 0)),
            pl.BlockSpec((D, 4 * H), lambda i: (0, 0)),
            pl.BlockSpec((H, 4 * H), lambda i: (0, 0)),
            pl.BlockSpec((1, 4 * H), lambda i: (0, 0)),
            pl.BlockSpec((1, 4 * H), lambda i: (0, 0)),
            pl.BlockSpec((K, BB, H), lambda i: (0, i, 0)),
            pl.BlockSpec((K, H, H), lambda i: (0, 0, 0)),
        ],
        out_specs=[
            pl.BlockSpec((BB, H), lambda i: (i, 0)),
            pl.BlockSpec((BB, H), lambda i: (i, 0)),
        ],
        out_shape=[
            jax.ShapeDtypeStruct((B, H), jnp.float32),
            jax.ShapeDtypeStruct((B, H), jnp.float32),
        ],
    )(x, hx, wih_t, whh_t, bih, bhh, cs, sel)


def _sc_winner_body(gx_hbm, gy_hbm, lw_hbm, gxv, gyv, lwin):
    wid = _worker_id()
    base = wid * BPW
    pltpu.sync_copy(gx_hbm.at[pl.ds(base, BPW)], gxv)
    pltpu.sync_copy(gy_hbm.at[pl.ds(base, BPW)], gyv)

    iota = jnp.arange(L, dtype=jnp.int32)
    neg1 = jnp.full((L,), -1, jnp.int32)

    def init_body(j, _):
        for u in range(4):
            lwin[pl.ds(j * 4 * L + u * L, L)] = neg1
        return 0

    lax.fori_loop(0, OUT_ROWS // (4 * L), init_body, 0)

    def group_body(g, _):
        s = pl.ds(g * L, L)
        cell16 = gxv[s] * NSIDE + gyv[s]
        for j in range(L):
            cell = cell16[j]
            wbase = pl.multiple_of((cell >> 4) << 4, L)
            lane = cell & (L - 1)
            win = lwin[pl.ds(wbase, L)]
            lwin[pl.ds(wbase, L)] = jnp.where(iota == lane, base + g * L + j, win)
        return 0

    lax.fori_loop(0, BPW // L, group_body, 0)
    pltpu.sync_copy(lwin, lw_hbm.at[wid])


def _sc_winner(gx, gy):
    fn = pl.kernel(
        _sc_winner_body,
        mesh=_mesh(),
        out_type=jax.ShapeDtypeStruct((NW, OUT_ROWS), jnp.int32),
        scratch_types=[
            pltpu.VMEM((BPW,), jnp.int32),
            pltpu.VMEM((BPW,), jnp.int32),
            pltpu.VMEM((OUT_ROWS,), jnp.int32),
        ],
    )
    return fn(gx, gy)


SNBUF = 4
SDEPTH = 3


def _sc_scatter_body(lw_hbm, comb_hbm, out_hbm, allw, allw2, acc, idxw, rows,
                     gsems, osems):
    wid = _worker_id()
    lo = wid * SPAN

    QH = 8
    mbufs = [allw, allw2]

    def mload(q):
        return pltpu.async_copy(lw_hbm.at[pl.ds(q * QH, QH), pl.ds(lo, SPAN)],
                                mbufs[q % 2], gsems[q % 2])

    mh = mload(0)
    for q in range(NW // QH):
        mh.wait()
        if q + 1 < NW // QH:
            mh = mload(q + 1)
        mb = mbufs[q % 2]

        def merge_j(j, _, mb=mb, first=(q == 0)):
            s = pl.ds(j * L, L)
            m = mb[0, s]
            for t in range(1, QH):
                m = jnp.maximum(m, mb[t, s])
            acc[s] = m if first else jnp.maximum(acc[s], m)
            return 0

        lax.fori_loop(0, SPAN // L, merge_j, 0)

    iota = jnp.arange(L, dtype=jnp.int32)

    def build_idx(w):
        for g in range(128 // L):
            c16 = lo + w * 128 + g * L + iota
            wv = acc[pl.ds(w * 128 + g * L, L)]
            src = jnp.where(wv >= 0, NN + wv, jnp.minimum(c16, NN - 1))
            idxw[pl.ds((w % SNBUF) * 128 + g * L, L)] = src

    def gissue(w):
        idxrow = idxw.at[pl.ds((w % SNBUF) * 128, 128)]
        return pltpu.async_copy(comb_hbm.at[idxrow], rows.at[w % SNBUF],
                                gsems[w % SNBUF])

    TAIL = NN - NN_ALIGN

    def omake(w):
        st = lo + w * 128
        slot = w % SNBUF
        main = pltpu.make_async_copy(
            rows.at[slot],
            out_hbm.at[pl.ds(pl.multiple_of(
                jnp.minimum(st, NN_ALIGN - 128), 128), 128), :],
            osems[slot])
        tail = pltpu.make_async_copy(
            rows.at[slot, pl.ds(0, TAIL), :],
            out_hbm.at[pl.ds(NN_ALIGN, TAIL), :],
            osems[slot])
        return st, main, tail

    def ostart(w):
        st, main, tail = omake(w)
        pl.when(st < NN_ALIGN)(main.start)
        pl.when(st == NN_ALIGN)(tail.start)

    def owait(w):
        st, main, tail = omake(w)
        pl.when(st < NN_ALIGN)(main.wait)
        pl.when(st == NN_ALIGN)(tail.wait)

    gh = {}
    owaited = set()
    for w in range(SDEPTH):
        build_idx(w)
        gh[w] = gissue(w)
    for w in range(NWIN):
        gh.pop(w).wait()
        ostart(w)
        u = w + SDEPTH
        if u < NWIN:
            if u >= SNBUF:
                owait(u - SNBUF)
                owaited.add(u - SNBUF)
            build_idx(u)
            gh[u] = gissue(u)
    for w in range(NWIN):
        if w not in owaited:
            owait(w)


def _sc_scatter(lw, combined):
    fn = pl.kernel(
        _sc_scatter_body,
        mesh=_mesh(),
        out_type=jax.ShapeDtypeStruct((NN, H), jnp.float32),
        scratch_types=[
            pltpu.VMEM((8, SPAN), jnp.int32),
            pltpu.VMEM((8, SPAN), jnp.int32),
            pltpu.VMEM((SPAN,), jnp.int32),
            pltpu.VMEM((SNBUF * 128,), jnp.int32),
            pltpu.VMEM((SNBUF, 128, H), jnp.float32),
            [pltpu.SemaphoreType.DMA] * SNBUF,
            [pltpu.SemaphoreType.DMA] * SNBUF,
        ],
    )
    return fn(lw, combined)


def kernel(input, hx, weight_ih, weight_hh, bias_ih, bias_hh, memory):
    coords = input[:, D:].astype(jnp.int32) + W
    gx = coords[:, 0]
    gy = coords[:, 1]
    mem2 = memory.reshape(NN, H)

    cs = _sc_gather(gx, gy, mem2)

    hyy, updates = _tc_main(
        input, hx, weight_ih.T, weight_hh.T,
        bias_ih.reshape(1, 4 * H), bias_hh.reshape(1, 4 * H), cs)

    combined = jnp.concatenate([mem2, updates], axis=0)
    lw = _sc_winner(gx, gy)
    outp = _sc_scatter(lw, combined)
    new_mem = outp.reshape(NSIDE, NSIDE, H)
    return hyy, new_mem


# speedup vs baseline: 16.7576x; 1.0014x over previous
import functools

import numpy as _np

import jax
import jax.numpy as jnp
from jax import lax
from jax.experimental import pallas as pl
from jax.experimental.pallas import tpu as pltpu
from jax.experimental.pallas import tpu_sc as plsc

W = 2
NSIDE = 262
NN = NSIDE * NSIDE
K = (2 * W + 1) ** 2
H = 128
D = 128
B = 16384

NC, NS, L = 2, 16, 16
NW = NC * NS
BPW = B // NW

OFFS = [dx * NSIDE + dy for dx in range(-W, W + 1) for dy in range(-W, W + 1)]

SPAN = 2176
NWIN = SPAN // 128
OUT_ROWS = NW * SPAN
NN_ALIGN = (NN // 128) * 128

def _mesh():
    return plsc.VectorSubcoreMesh(core_axis_name="c", subcore_axis_name="s")


def _worker_id():
    return lax.axis_index("s") * NC + lax.axis_index("c")


GWIN = 128
GNBUF = 6
GDEPTH = 4


def _sc_gather_body(gx_hbm, gy_hbm, mem2_hbm, out_hbm, gxv, gyv, idx2, rows,
                    gsems, osems):
    wid = _worker_id()
    base = wid * BPW
    pltpu.sync_copy(gx_hbm.at[pl.ds(base, BPW)], gxv)
    pltpu.sync_copy(gy_hbm.at[pl.ds(base, BPW)], gyv)

    def idx_body(g, _):
        s = pl.ds(g * L, L)
        f = gxv[s] * NSIDE + gyv[s]
        for k in range(K):
            idx2[pl.ds(k * BPW + g * L, L)] = f + OFFS[k]
        return 0

    lax.fori_loop(0, BPW // L, idx_body, 0)

    wpk = BPW // GWIN
    nt = K * wpk

    def gissue(t):
        k, w = divmod(t, wpk)
        idxrow = idx2.at[pl.ds(k * BPW + w * GWIN, GWIN)]
        return pltpu.async_copy(mem2_hbm.at[idxrow], rows.at[t % GNBUF],
                                gsems[t % GNBUF])

    def oissue(t):
        k, w = divmod(t, wpk)
        return pltpu.async_copy(rows.at[t % GNBUF],
                                out_hbm.at[k, pl.ds(base + w * GWIN, GWIN), :],
                                osems[t % GNBUF])

    gh = {t: gissue(t) for t in range(GDEPTH)}
    oh = {}
    for t in range(nt):
        gh.pop(t).wait()
        oh[t] = oissue(t)
        u = t + GDEPTH
        if u < nt:
            if u >= GNBUF:
                oh.pop(u - GNBUF).wait()
            gh[u] = gissue(u)
    for t in sorted(oh):
        oh[t].wait()


def _sc_gather(gx, gy, mem2):
    fn = pl.kernel(
        _sc_gather_body,
        mesh=_mesh(),
        out_type=jax.ShapeDtypeStruct((K, B, H), jnp.float32),
        scratch_types=[
            pltpu.VMEM((BPW,), jnp.int32),
            pltpu.VMEM((BPW,), jnp.int32),
            pltpu.VMEM((K * BPW,), jnp.int32),
            pltpu.VMEM((GNBUF, GWIN, H), jnp.float32),
            [pltpu.SemaphoreType.DMA] * GNBUF,
            [pltpu.SemaphoreType.DMA] * GNBUF,
        ],
    )
    return fn(gx, gy, mem2)


BB = 1024


def _tc_body(x_ref, hx_ref, wih_ref, whh_ref, bih_ref, bhh_ref, cs_ref,
             sel_ref, hyy_ref, upd_ref):
    xg = jnp.dot(x_ref[:, :D], wih_ref[...],
                 preferred_element_type=jnp.float32) + bih_ref[...]
    hg = jnp.dot(hx_ref[...], whh_ref[...],
                 preferred_element_type=jnp.float32) + bhh_ref[...]
    r = jax.nn.sigmoid(xg[:, :H] + hg[:, :H])
    u = jax.nn.sigmoid(xg[:, H:2 * H] + hg[:, H:2 * H])
    sg = jax.nn.sigmoid(xg[:, 3 * H:] + hg[:, 3 * H:])
    ng = jnp.tanh(xg[:, 2 * H:3 * H] + r * hg[:, 2 * H:3 * H])

    scores = jnp.concatenate(
        [jnp.sum(cs_ref[k] * ng, axis=1, keepdims=True) for k in range(K)],
        axis=1)
    m = jnp.max(scores, axis=1, keepdims=True)
    e = jnp.exp(scores - m)
    inv = 1.0 / jnp.sum(e, axis=1, keepdims=True)
    attn = jnp.pad(e * inv, ((0, 0), (0, H - K)))
    acc = jnp.zeros((BB, H), jnp.float32)
    for k in range(K):
        ak = jnp.dot(attn, sel_ref[k], preferred_element_type=jnp.float32)
        acc = acc + ak * cs_ref[k]
    curr = ng + sg * acc
    hyy = curr + u * (hx_ref[...] - curr)
    hyy_ref[...] = hyy
    upd_ref[...] = sg * cs_ref[W * (2 * W + 1) + W] + (1.0 - sg) * hyy


def _tc_main(x, hx, wih_t, whh_t, bih, bhh, cs):
    grid = (B // BB,)
    sel = jnp.asarray(
        (_np.arange(H)[None, :, None] == _np.arange(K)[:, None, None])
        .astype(_np.float32) * _np.ones((1, 1, H), _np.float32))
    return pl.pallas_call(
        _tc_body,
        grid=grid,
        in_specs=[
            pl.BlockSpec((BB, D + 2), lambda i: (i, 0)),
            pl.BlockSpec((BB, H), lambda i: (i, 0)),
            pl.BlockSpec((D, 4 * H), lambda i: (0, 0)),
            pl.BlockSpec((H, 4 * H), lambda i: (0, 0)),
            pl.BlockSpec((1, 4 * H), lambda i: (0, 0)),
            pl.BlockSpec((1, 4 * H), lambda i: (0, 0)),
            pl.BlockSpec((K, BB, H), lambda i: (0, i, 0)),
            pl.BlockSpec((K, H, H), lambda i: (0, 0, 0)),
        ],
        out_specs=[
            pl.BlockSpec((BB, H), lambda i: (i, 0)),
            pl.BlockSpec((BB, H), lambda i: (i, 0)),
        ],
        out_shape=[
            jax.ShapeDtypeStruct((B, H), jnp.float32),
            jax.ShapeDtypeStruct((B, H), jnp.float32),
        ],
    )(x, hx, wih_t, whh_t, bih, bhh, cs, sel)


def _sc_winner_body(gx_hbm, gy_hbm, lw_hbm, gxv, gyv, lwin):
    wid = _worker_id()
    base = wid * BPW
    pltpu.sync_copy(gx_hbm.at[pl.ds(base, BPW)], gxv)
    pltpu.sync_copy(gy_hbm.at[pl.ds(base, BPW)], gyv)

    iota = jnp.arange(L, dtype=jnp.int32)
    neg1 = jnp.full((L,), -1, jnp.int32)

    def init_body(j, _):
        for u in range(4):
            lwin[pl.ds(j * 4 * L + u * L, L)] = neg1
        return 0

    lax.fori_loop(0, OUT_ROWS // (4 * L), init_body, 0)

    def group_body(g, _):
        s = pl.ds(g * L, L)
        cell16 = gxv[s] * NSIDE + gyv[s]
        for j in range(L):
            cell = cell16[j]
            wbase = pl.multiple_of((cell >> 4) << 4, L)
            lane = cell & (L - 1)
            win = lwin[pl.ds(wbase, L)]
            lwin[pl.ds(wbase, L)] = jnp.where(iota == lane, base + g * L + j, win)
        return 0

    lax.fori_loop(0, BPW // L, group_body, 0)
    pltpu.sync_copy(lwin, lw_hbm.at[wid])


def _sc_winner(gx, gy):
    fn = pl.kernel(
        _sc_winner_body,
        mesh=_mesh(),
        out_type=jax.ShapeDtypeStruct((NW, OUT_ROWS), jnp.int32),
        scratch_types=[
            pltpu.VMEM((BPW,), jnp.int32),
            pltpu.VMEM((BPW,), jnp.int32),
            pltpu.VMEM((OUT_ROWS,), jnp.int32),
        ],
    )
    return fn(gx, gy)


SNBUF = 4
SDEPTH = 3


def _sc_scatter_body(lw_hbm, comb_hbm, out_hbm, allw, allw2, acc, idxw, rows,
                     gsems, osems):
    wid = _worker_id()
    lo = wid * SPAN

    QH = 8
    mbufs = [allw, allw2]

    def mload(q):
        return pltpu.async_copy(lw_hbm.at[pl.ds(q * QH, QH), pl.ds(lo, SPAN)],
                                mbufs[q % 2], gsems[q % 2])

    mh = mload(0)
    for q in range(NW // QH):
        mh.wait()
        if q + 1 < NW // QH:
            mh = mload(q + 1)
        mb = mbufs[q % 2]

        def merge_j(j, _, mb=mb, first=(q == 0)):
            s = pl.ds(j * L, L)
            m = mb[0, s]
            for t in range(1, QH):
                m = jnp.maximum(m, mb[t, s])
            acc[s] = m if first else jnp.maximum(acc[s], m)
            return 0

        lax.fori_loop(0, SPAN // L, merge_j, 0)

    iota = jnp.arange(L, dtype=jnp.int32)

    def build_idx(w):
        for g in range(128 // L):
            c16 = lo + w * 128 + g * L + iota
            wv = acc[pl.ds(w * 128 + g * L, L)]
            src = jnp.where(wv >= 0, NN + wv, jnp.minimum(c16, NN - 1))
            idxw[pl.ds((w % SNBUF) * 128 + g * L, L)] = src

    def gissue(w):
        idxrow = idxw.at[pl.ds((w % SNBUF) * 128, 128)]
        return pltpu.async_copy(comb_hbm.at[idxrow], rows.at[w % SNBUF],
                                gsems[w % SNBUF])

    TAIL = NN - NN_ALIGN

    def omake(w):
        st = lo + w * 128
        slot = w % SNBUF
        main = pltpu.make_async_copy(
            rows.at[slot],
            out_hbm.at[pl.ds(pl.multiple_of(
                jnp.minimum(st, NN_ALIGN - 128), 128), 128), :],
            osems[slot])
        tail = pltpu.make_async_copy(
            rows.at[slot, pl.ds(0, TAIL), :],
            out_hbm.at[pl.ds(NN_ALIGN, TAIL), :],
            osems[slot])
        return st, main, tail

    def ostart(w):
        st, main, tail = omake(w)
        pl.when(st < NN_ALIGN)(main.start)
        pl.when(st == NN_ALIGN)(tail.start)

    def owait(w):
        st, main, tail = omake(w)
        pl.when(st < NN_ALIGN)(main.wait)
        pl.when(st == NN_ALIGN)(tail.wait)

    gh = {}
    owaited = set()
    for w in range(SDEPTH):
        build_idx(w)
        gh[w] = gissue(w)
    for w in range(NWIN):
        gh.pop(w).wait()
        ostart(w)
        u = w + SDEPTH
        if u < NWIN:
            if u >= SNBUF:
                owait(u - SNBUF)
                owaited.add(u - SNBUF)
            build_idx(u)
            gh[u] = gissue(u)
    for w in range(NWIN):
        if w not in owaited:
            owait(w)


def _sc_scatter(lw, combined):
    fn = pl.kernel(
        _sc_scatter_body,
        mesh=_mesh(),
        out_type=jax.ShapeDtypeStruct((NN, H), jnp.float32),
        scratch_types=[
            pltpu.VMEM((8, SPAN), jnp.int32),
            pltpu.VMEM((8, SPAN), jnp.int32),
            pltpu.VMEM((SPAN,), jnp.int32),
            pltpu.VMEM((SNBUF * 128,), jnp.int32),
            pltpu.VMEM((SNBUF, 128, H), jnp.float32),
            [pltpu.SemaphoreType.DMA] * SNBUF,
            [pltpu.SemaphoreType.DMA] * SNBUF,
        ],
    )
    return fn(lw, combined)


def kernel(input, hx, weight_ih, weight_hh, bias_ih, bias_hh, memory):
    coords = input[:, D:].astype(jnp.int32) + W
    gx = coords[:, 0]
    gy = coords[:, 1]
    mem2 = memory.reshape(NN, H)

    cs = _sc_gather(gx, gy, mem2)

    hyy, updates = _tc_main(
        input, hx, weight_ih.T, weight_hh.T,
        bias_ih.reshape(1, 4 * H), bias_hh.reshape(1, 4 * H), cs)

    combined = jnp.concatenate([mem2, updates], axis=0)
    lw = _sc_winner(gx, gy)
    outp = _sc_scatter(lw, combined)
    new_mem = outp.reshape(NSIDE, NSIDE, H)
    return hyy, new_mem
